# baseline (speedup 1.0000x reference)
import jax
import jax.numpy as jnp
from jax import lax
from jax.experimental import pallas as pl

_MASK_PERCENT = 0.5


def _body(r_ref, x_ref, o_ref):
    freq = r_ref.shape[0]
    n2 = r_ref.shape[-1]
    keep_thresh = float(int(_MASK_PERCENT * freq))
    r0 = r_ref[:, 0, 0, :]
    bits = lax.bitcast_convert_type(r0, jnp.int32)
    fidx = lax.broadcasted_iota(jnp.int32, (freq, n2), 0)
    key = (bits & jnp.int32(~63)) | fidx
    less = key[None, :, :] < key[:, None, :]
    rank = jnp.sum(less.astype(jnp.float32), axis=1)
    m = (rank >= keep_thresh).astype(jnp.float32)
    o_ref[...] = x_ref[...] * m[:, None, None, :]


def kernel(x):
    freq, batch, n1, n2, length = x.shape
    rkey = jax.random.key(42)
    r = jax.random.uniform(rkey, (freq, batch, n1, n2), dtype=jnp.float32)
    r4 = r.reshape(freq, batch * n1, 1, n2)
    xt = jnp.transpose(x, (0, 1, 2, 4, 3))
    x4 = xt.reshape(freq, batch * n1, length, n2)
    grid = batch * n1
    out = pl.pallas_call(
        _body,
        grid=(grid,),
        in_specs=[
            pl.BlockSpec((freq, 1, 1, n2), lambda g: (0, g, 0, 0)),
            pl.BlockSpec((freq, 1, length, n2), lambda g: (0, g, 0, 0)),
        ],
        out_specs=pl.BlockSpec((freq, 1, length, n2), lambda g: (0, g, 0, 0)),
        out_shape=jax.ShapeDtypeStruct((freq, batch * n1, length, n2),
                                       jnp.float32),
    )(r4, x4)
    out5 = out.reshape(freq, batch, n1, length, n2)
    return jnp.transpose(out5, (0, 1, 2, 4, 3))


# speedup vs baseline: 6.9315x; 6.9315x over previous
import jax
import jax.numpy as jnp
from jax import lax
from jax.experimental import pallas as pl

_MASK_PERCENT = 0.5


def _body(r_ref, x_ref, o_ref):
    freq, cb = r_ref.shape
    n2 = x_ref.shape[-1]
    npair = x_ref.shape[1]
    keep_thresh = float(int(_MASK_PERCENT * freq))
    bits = lax.bitcast_convert_type(r_ref[...], jnp.int32)
    fidx = lax.broadcasted_iota(jnp.int32, (freq, cb), 0)
    key = (bits & jnp.int32(~63)) | fidx
    less = key[None, :, :] < key[:, None, :]
    rank = jnp.sum(less.astype(jnp.float32), axis=1)
    m = (rank >= keep_thresh).astype(jnp.float32)
    for i in range(npair):
        mi = m[:, i * n2:(i + 1) * n2]
        o_ref[:, i] = x_ref[:, i] * mi[:, None, :]


def kernel(x):
    freq, batch, n1, n2, length = x.shape
    ncols = batch * n1 * n2
    rkey = jax.random.key(42)
    r2 = jax.random.uniform(rkey, (freq, ncols), dtype=jnp.float32)
    xt = jnp.transpose(x, (0, 1, 2, 4, 3))
    x4 = xt.reshape(freq, batch * n1, length, n2)
    npair = 128 // n2
    grid = (batch * n1) // npair
    out = pl.pallas_call(
        _body,
        grid=(grid,),
        in_specs=[
            pl.BlockSpec((freq, npair * n2), lambda g: (0, g)),
            pl.BlockSpec((freq, npair, length, n2), lambda g: (0, g, 0, 0)),
        ],
        out_specs=pl.BlockSpec((freq, npair, length, n2),
                               lambda g: (0, g, 0, 0)),
        out_shape=jax.ShapeDtypeStruct((freq, batch * n1, length, n2),
                                       jnp.float32),
    )(r2, x4)
    out5 = out.reshape(freq, batch, n1, length, n2)
    return jnp.transpose(out5, (0, 1, 2, 4, 3))
